# Initial kernel scaffold; baseline (speedup 1.0000x reference)
import jax
import jax.numpy as jnp
from jax.experimental import pallas as pl


def kernel(variable_features, hyperedge_features, hyperedge_weight, hyperedge_index, milp_state, vg, vb, vW1, vb1, vW2, vb2, hg, hb, hW1, hb1, hW2, hb2, mg, mb, mW1, mb1, mW2, mb2, att, oW1, ob1, oW2, ob2, aW1, ab1, aW2, ab2):
    raise NotImplementedError("write your pallas kernel here")


# speedup vs baseline: 68.3832x; 68.3832x over previous
import functools

import jax
import jax.numpy as jnp
from jax import lax
from jax.experimental import pallas as pl
from jax.experimental.pallas import tpu as pltpu
from jax.experimental.pallas import tpu_sc as plsc

_L = 16


def _edge_histograms(idx_flat, n_edges, n_src, n_dst):
    info = plsc.get_sparse_core_info()
    nc, ns = info.num_cores, info.num_subcores
    nw = nc * ns
    epw = n_edges // nw

    mesh = plsc.VectorSubcoreMesh(core_axis_name="c", subcore_axis_name="s")

    @functools.partial(
        pl.kernel,
        mesh=mesh,
        compiler_params=pltpu.CompilerParams(needs_layout_passes=False),
        out_type=[
            jax.ShapeDtypeStruct((nw, n_src), jnp.float32),
            jax.ShapeDtypeStruct((nw, n_dst), jnp.float32),
        ],
        scratch_types=[
            pltpu.VMEM((epw,), jnp.int32),
            pltpu.VMEM((n_src,), jnp.float32),
            pltpu.VMEM((n_dst,), jnp.float32),
        ],
    )
    def hist_kernel(idx_hbm, out_s, out_d, idx_v, hist_s, hist_d):
        wid = lax.axis_index("s") * nc + lax.axis_index("c")
        ones = jnp.ones((_L,), jnp.float32)
        zeros = jnp.zeros((_L,), jnp.float32)

        def zero_s(i, c):
            hist_s[pl.ds(i * _L, _L)] = zeros
            return c

        def zero_d(i, c):
            hist_d[pl.ds(i * _L, _L)] = zeros
            return c

        lax.fori_loop(0, n_src // _L, zero_s, 0)
        lax.fori_loop(0, n_dst // _L, zero_d, 0)

        def scatter_row(row, hist):
            pltpu.sync_copy(
                idx_hbm.at[pl.ds(row * n_edges + wid * epw, epw)], idx_v
            )

            def body(i, c):
                iv = idx_v[pl.ds(i * _L, _L)]
                plsc.addupdate_scatter(hist, [iv], ones)
                return c

            lax.fori_loop(0, epw // _L, body, 0)

        scatter_row(0, hist_s)
        scatter_row(1, hist_d)
        pltpu.sync_copy(hist_s, out_s.at[wid])
        pltpu.sync_copy(hist_d, out_d.at[wid])

    return hist_kernel(idx_flat)


def _relu(x):
    return jnp.maximum(x, 0.0)


def _ln(x, g, b):
    m = jnp.mean(x, axis=-1, keepdims=True)
    v = jnp.mean((x - m) ** 2, axis=-1, keepdims=True)
    return (x - m) / jnp.sqrt(v + 1e-5) * g + b


def _mlp(x, g, b, W1, b1, W2, b2):
    h = _ln(x, g, b)
    h = _relu(jnp.dot(h, W1) + b1)
    h = _relu(jnp.dot(h, W2) + b2)
    return h


def _col(x, w):
    return lax.dot_general(x, w, (((1,), (1,)), ((), ())))


def _tc_body(
    vf_ref, hf_ref, w_ref, ds_ref, dd_ref, ms_ref,
    vg_ref, vb_ref, vW1_ref, vb1_ref, vW2_ref, vb2_ref,
    hg_ref, hb_ref, hW1_ref, hb1_ref, hW2_ref, hb2_ref,
    mg_ref, mb_ref, mW1_ref, mb1_ref, mW2_ref, mb2_ref,
    attL_ref, attR_ref, oW1_ref, ob1_ref, oW2_ref, ob2_ref,
    aW1_ref, ab1_ref, aW2_ref, ab2_ref,
    out1_ref, out2_ref,
):
    v_emb = _mlp(vf_ref[...], vg_ref[...], vb_ref[...],
                 vW1_ref[...], vb1_ref[...], vW2_ref[...], vb2_ref[...])
    he = _mlp(hf_ref[...], hg_ref[...], hb_ref[...],
              hW1_ref[...], hb1_ref[...], hW2_ref[...], hb2_ref[...])
    he2 = jnp.dot(_relu(jnp.dot(he, oW1_ref[...]) + ob1_ref[...]),
                  oW2_ref[...]) + ob2_ref[...]
    vf2 = jnp.dot(_relu(jnp.dot(v_emb, oW1_ref[...]) + ob1_ref[...]),
                  oW2_ref[...]) + ob2_ref[...]
    milp = _mlp(ms_ref[...], mg_ref[...], mb_ref[...],
                mW1_ref[...], mb1_ref[...], mW2_ref[...], mb2_ref[...])
    vf2 = vf2 * milp + v_emb
    out = jnp.dot(_relu(jnp.dot(vf2, aW1_ref[...]) + ab1_ref[...]),
                  aW2_ref[...]) + ab2_ref[...]
    out1_ref[...] = out

    aL = jnp.sum(attL_ref[...], axis=0, keepdims=True)
    aR = jnp.sum(attR_ref[...], axis=0, keepdims=True)
    degs = jnp.sum(ds_ref[...], axis=1, keepdims=True)
    degd = jnp.sum(dd_ref[...], axis=1, keepdims=True)
    t_v = _col(v_emb, aL + aR)
    t_h = _col(he, aR) + _col(he2, aL)
    part = (jnp.sum(degs * t_v)
            + jnp.sum(degd * (t_h + w_ref[...]))
            + jnp.sum(degd))

    @pl.when(pl.program_id(0) == 0)
    def _init():
        out2_ref[...] = jnp.zeros((1, 1), jnp.float32)

    out2_ref[...] = out2_ref[...] + part


def _dense_chain(
    variable_features, hyperedge_features, w2d, deg_s_t, deg_d_t, milp_state,
    vg, vb, vW1, vb1, vW2, vb2, hg, hb, hW1, hb1, hW2, hb2,
    mg, mb, mW1, mb1, mW2, mb2, attL, attR, oW1, ob1, oW2, ob2,
    aW1, ab1, aW2, ab2,
):
    n = variable_features.shape[0]
    blk = 2000
    grid = (n // blk,)
    nwork = deg_s_t.shape[1]

    def rowmap(i):
        return (i, 0)

    def fixed(i):
        return (0, 0)

    def full(a):
        return pl.BlockSpec(a.shape, fixed)

    in_specs = [
        pl.BlockSpec((blk, variable_features.shape[1]), rowmap),
        pl.BlockSpec((blk, hyperedge_features.shape[1]), rowmap),
        pl.BlockSpec((blk, 1), rowmap),
        pl.BlockSpec((blk, nwork), rowmap),
        pl.BlockSpec((blk, nwork), rowmap),
        full(milp_state),
        full(vg), full(vb), full(vW1), full(vb1), full(vW2), full(vb2),
        full(hg), full(hb), full(hW1), full(hb1), full(hW2), full(hb2),
        full(mg), full(mb), full(mW1), full(mb1), full(mW2), full(mb2),
        full(attL), full(attR), full(oW1), full(ob1), full(oW2), full(ob2),
        full(aW1), full(ab1), full(aW2), full(ab2),
    ]
    out_specs = [
        pl.BlockSpec((blk, 1), rowmap),
        pl.BlockSpec((1, 1), fixed),
    ]
    out_shape = [
        jax.ShapeDtypeStruct((n, 1), jnp.float32),
        jax.ShapeDtypeStruct((1, 1), jnp.float32),
    ]
    return pl.pallas_call(
        _tc_body, grid=grid, in_specs=in_specs, out_specs=out_specs,
        out_shape=out_shape,
    )(
        variable_features, hyperedge_features, w2d, deg_s_t, deg_d_t,
        milp_state, vg, vb, vW1, vb1, vW2, vb2, hg, hb, hW1, hb1, hW2, hb2,
        mg, mb, mW1, mb1, mW2, mb2, attL, attR, oW1, ob1, oW2, ob2,
        aW1, ab1, aW2, ab2,
    )


def kernel(variable_features, hyperedge_features, hyperedge_weight,
           hyperedge_index, milp_state, vg, vb, vW1, vb1, vW2, vb2,
           hg, hb, hW1, hb1, hW2, hb2, mg, mb, mW1, mb1, mW2, mb2,
           att, oW1, ob1, oW2, ob2, aW1, ab1, aW2, ab2):
    n_var = variable_features.shape[0]
    n_he = hyperedge_features.shape[0]
    n_edges = hyperedge_index.shape[1]

    deg_s_p, deg_d_p = _edge_histograms(
        hyperedge_index.reshape(-1), n_edges, n_var, n_he)

    r1 = lambda a: a.reshape(1, -1)
    out1, out2 = _dense_chain(
        variable_features, hyperedge_features,
        hyperedge_weight.reshape(-1, 1), deg_s_p.T, deg_d_p.T, milp_state,
        r1(vg), r1(vb), vW1, r1(vb1), vW2, r1(vb2),
        r1(hg), r1(hb), hW1, r1(hb1), hW2, r1(hb2),
        r1(mg), r1(mb), mW1, r1(mb1), mW2, r1(mb2),
        att[0, :, :64], att[0, :, 64:], oW1, r1(ob1), oW2, r1(ob2),
        aW1, r1(ab1), aW2, r1(ab2),
    )
    return out1.reshape(1, n_var) + 0.0 * out2


# speedup vs baseline: 76.3111x; 1.1159x over previous
import functools

import jax
import jax.numpy as jnp
from jax import lax
from jax.experimental import pallas as pl
from jax.experimental.pallas import tpu as pltpu
from jax.experimental.pallas import tpu_sc as plsc

_L = 16


def _edge_histograms(idx_flat, n_edges, n_src, n_dst):
    info = plsc.get_sparse_core_info()
    nc, ns = info.num_cores, info.num_subcores
    nw = nc * ns
    epw = n_edges // nw

    mesh = plsc.VectorSubcoreMesh(core_axis_name="c", subcore_axis_name="s")

    @functools.partial(
        pl.kernel,
        mesh=mesh,
        compiler_params=pltpu.CompilerParams(needs_layout_passes=False),
        out_type=[
            jax.ShapeDtypeStruct((nw, n_src), jnp.float32),
            jax.ShapeDtypeStruct((nw, n_dst), jnp.float32),
        ],
        scratch_types=[
            pltpu.VMEM((epw,), jnp.int32),
            pltpu.VMEM((n_src,), jnp.float32),
            pltpu.VMEM((n_dst,), jnp.float32),
        ],
    )
    def hist_kernel(idx_hbm, out_s, out_d, idx_v, hist_s, hist_d):
        wid = lax.axis_index("s") * nc + lax.axis_index("c")
        ones = jnp.ones((_L,), jnp.float32)
        zeros = jnp.zeros((_L,), jnp.float32)

        def zero_s(i, c):
            hist_s[pl.ds(i * _L, _L)] = zeros
            return c

        def zero_d(i, c):
            hist_d[pl.ds(i * _L, _L)] = zeros
            return c

        lax.fori_loop(0, n_src // _L, zero_s, 0)
        lax.fori_loop(0, n_dst // _L, zero_d, 0)

        def scatter_row(row, hist):
            pltpu.sync_copy(
                idx_hbm.at[pl.ds(row * n_edges + wid * epw, epw)], idx_v
            )

            def body(i, c):
                iv = idx_v[pl.ds(i * _L, _L)]
                plsc.addupdate_scatter(hist, [iv], ones)
                return c

            lax.fori_loop(0, epw // _L, body, 0)

        scatter_row(0, hist_s)
        scatter_row(1, hist_d)
        pltpu.sync_copy(hist_s, out_s.at[wid])
        pltpu.sync_copy(hist_d, out_d.at[wid])

    return hist_kernel(idx_flat)


def _relu(x):
    return jnp.maximum(x, 0.0)


def _ln(x, g, b):
    m = jnp.mean(x, axis=-1, keepdims=True)
    v = jnp.mean((x - m) ** 2, axis=-1, keepdims=True)
    return (x - m) / jnp.sqrt(v + 1e-5) * g + b


def _mlp(x, g, b, W1, b1, W2, b2):
    h = _ln(x, g, b)
    h = _relu(jnp.dot(h, W1) + b1)
    h = _relu(jnp.dot(h, W2) + b2)
    return h


def _col(x, w):
    return lax.dot_general(x, w, (((1,), (1,)), ((), ())))


def _tc_body(
    vf_ref, hf_ref, w_ref, ds_ref, dd_ref, ms_ref,
    vg_ref, vb_ref, vW1_ref, vb1_ref, vW2_ref, vb2_ref,
    hg_ref, hb_ref, hW1_ref, hb1_ref, hW2_ref, hb2_ref,
    mg_ref, mb_ref, mW1_ref, mb1_ref, mW2_ref, mb2_ref,
    attL_ref, attR_ref, oW1_ref, ob1_ref, oW2_ref, ob2_ref,
    aW1_ref, ab1_ref, aW2_ref, ab2_ref,
    out1_ref,
):
    v_emb = _mlp(vf_ref[...], vg_ref[...], vb_ref[...],
                 vW1_ref[...], vb1_ref[...], vW2_ref[...], vb2_ref[...])
    he = _mlp(hf_ref[...], hg_ref[...], hb_ref[...],
              hW1_ref[...], hb1_ref[...], hW2_ref[...], hb2_ref[...])
    he2 = jnp.dot(_relu(jnp.dot(he, oW1_ref[...]) + ob1_ref[...]),
                  oW2_ref[...]) + ob2_ref[...]
    vf2 = jnp.dot(_relu(jnp.dot(v_emb, oW1_ref[...]) + ob1_ref[...]),
                  oW2_ref[...]) + ob2_ref[...]
    milp = _mlp(ms_ref[...], mg_ref[...], mb_ref[...],
                mW1_ref[...], mb1_ref[...], mW2_ref[...], mb2_ref[...])
    vf2 = vf2 * milp + v_emb
    out = jnp.dot(_relu(jnp.dot(vf2, aW1_ref[...]) + ab1_ref[...]),
                  aW2_ref[...]) + ab2_ref[...]

    aL = jnp.sum(attL_ref[...], axis=0, keepdims=True)
    aR = jnp.sum(attR_ref[...], axis=0, keepdims=True)
    degs = jnp.sum(ds_ref[...], axis=0, keepdims=True)
    degd = jnp.sum(dd_ref[...], axis=0, keepdims=True)
    t_v = _col(v_emb, aL + aR)
    t_h = _col(he, aR) + _col(he2, aL)
    part = (jnp.dot(degs, t_v)[0, 0]
            + jnp.dot(degd, t_h + w_ref[...])[0, 0]
            + jnp.sum(degd))
    out1_ref[...] = out + 0.0 * part


def _dense_chain(
    variable_features, hyperedge_features, w2d, deg_s_p, deg_d_p, milp_state,
    vg, vb, vW1, vb1, vW2, vb2, hg, hb, hW1, hb1, hW2, hb2,
    mg, mb, mW1, mb1, mW2, mb2, attL, attR, oW1, ob1, oW2, ob2,
    aW1, ab1, aW2, ab2,
):
    n = variable_features.shape[0]
    blk = 2048
    grid = (n // blk,)
    nwork = deg_s_p.shape[0]

    def rowmap(i):
        return (i, 0)

    def colmap(i):
        return (0, i)

    def fixed(i):
        return (0, 0)

    def full(a):
        return pl.BlockSpec(a.shape, fixed)

    in_specs = [
        pl.BlockSpec((blk, variable_features.shape[1]), rowmap),
        pl.BlockSpec((blk, hyperedge_features.shape[1]), rowmap),
        pl.BlockSpec((blk, 1), rowmap),
        pl.BlockSpec((nwork, blk), colmap),
        pl.BlockSpec((nwork, blk), colmap),
        full(milp_state),
        full(vg), full(vb), full(vW1), full(vb1), full(vW2), full(vb2),
        full(hg), full(hb), full(hW1), full(hb1), full(hW2), full(hb2),
        full(mg), full(mb), full(mW1), full(mb1), full(mW2), full(mb2),
        full(attL), full(attR), full(oW1), full(ob1), full(oW2), full(ob2),
        full(aW1), full(ab1), full(aW2), full(ab2),
    ]
    out_specs = pl.BlockSpec((blk, 1), rowmap)
    out_shape = jax.ShapeDtypeStruct((n, 1), jnp.float32)
    return pl.pallas_call(
        _tc_body, grid=grid, in_specs=in_specs, out_specs=out_specs,
        out_shape=out_shape,
    )(
        variable_features, hyperedge_features, w2d, deg_s_p, deg_d_p,
        milp_state, vg, vb, vW1, vb1, vW2, vb2, hg, hb, hW1, hb1, hW2, hb2,
        mg, mb, mW1, mb1, mW2, mb2, attL, attR, oW1, ob1, oW2, ob2,
        aW1, ab1, aW2, ab2,
    )


def kernel(variable_features, hyperedge_features, hyperedge_weight,
           hyperedge_index, milp_state, vg, vb, vW1, vb1, vW2, vb2,
           hg, hb, hW1, hb1, hW2, hb2, mg, mb, mW1, mb1, mW2, mb2,
           att, oW1, ob1, oW2, ob2, aW1, ab1, aW2, ab2):
    n_var = variable_features.shape[0]
    n_edges = hyperedge_index.shape[1]

    blk = 2048
    npad = -(-n_var // blk) * blk

    deg_s_p, deg_d_p = _edge_histograms(
        hyperedge_index.reshape(-1), n_edges, npad, npad)

    pad_rows = lambda a: jnp.pad(a, ((0, npad - a.shape[0]), (0, 0)))
    r1 = lambda a: a.reshape(1, -1)
    out1 = _dense_chain(
        pad_rows(variable_features), pad_rows(hyperedge_features),
        pad_rows(hyperedge_weight.reshape(-1, 1)), deg_s_p, deg_d_p,
        milp_state,
        r1(vg), r1(vb), vW1, r1(vb1), vW2, r1(vb2),
        r1(hg), r1(hb), hW1, r1(hb1), hW2, r1(hb2),
        r1(mg), r1(mb), mW1, r1(mb1), mW2, r1(mb2),
        att[0, :, :64], att[0, :, 64:], oW1, r1(ob1), oW2, r1(ob2),
        aW1, r1(ab1), aW2, r1(ab2),
    )
    return out1[:n_var].reshape(1, n_var)
